# baseline (speedup 1.0000x reference)
import functools

import jax
import jax.numpy as jnp
from jax.experimental import pallas as pl
from jax.experimental.pallas import tpu as pltpu

_SUB = 16
_NCHUNKS = 1

_LOG2E = 1.4426950408889634


def _trunk_kernel(gcat_ref, scat_ref, sraw_ref, draw_ref, dmu_ref, se_ref,
                  u0c0_ref, w12_ref, abond_ref, cbond_ref,
                  fw1_ref, fb1_ref, fw2_ref, fb2_ref, fw3t_ref, fb3_ref,
                  out_ref, *, n_layers, n_sites, n_edges, sub, nchunks):
    S = se_ref.shape[1]
    S2 = 2 * S
    N, E = n_sites, n_edges
    bf16, f32 = jnp.bfloat16, jnp.float32
    csub = sub // nchunks

    gcat = gcat_ref[...]
    scat = scat_ref[...]
    dmu = dmu_ref[...]
    se = se_ref[...]
    u0c0 = u0c0_ref[...]

    outs = []
    for h in range(nchunks):
        x2 = sraw_ref[0, h * csub * N:(h + 1) * csub * N, :]
        d2 = draw_ref[0, h * csub * E:(h + 1) * csub * E, :]
        s = jnp.dot(x2, se, preferred_element_type=f32)
        t = jnp.dot(d2, dmu, preferred_element_type=f32)
        gb = jnp.exp2(jnp.square(t) * (-_LOG2E))
        zb = jnp.dot(gb.astype(bf16), abond_ref[...],
                     preferred_element_type=f32) + cbond_ref[...]

        for l in range(n_layers):
            if l == 0:
                proj = jnp.dot(x2, u0c0,
                               preferred_element_type=f32).astype(bf16)
            else:
                proj = jnp.dot(s.astype(bf16), w12_ref[l],
                               preferred_element_type=f32).astype(bf16)
            p = jnp.concatenate(
                [jnp.concatenate([proj[b * N:(b + 1) * N, :S2],
                                  proj[b * N:(b + 1) * N, S2:]], axis=0)
                 for b in range(csub)], axis=1)
            z_all = jnp.dot(gcat, p, preferred_element_type=f32)
            zr = jnp.concatenate([z_all[:, b * S2:(b + 1) * S2]
                                  for b in range(csub)], axis=0)
            z = zr + zb[:, l * S2:(l + 1) * S2]
            sig = pl.reciprocal(1.0 + jnp.exp2(z * (-_LOG2E)), approx=False)
            v = sig[:, :S] * jnp.maximum(z[:, S:], 0.0)
            vcat = jnp.concatenate([v[b * E:(b + 1) * E]
                                    for b in range(csub)],
                                   axis=1).astype(bf16)
            delta = jnp.dot(scat, vcat, preferred_element_type=f32)
            s = s + jnp.concatenate([delta[:, b * S:(b + 1) * S]
                                     for b in range(csub)], axis=0)
        pooled = jnp.mean(s.reshape(csub, N, S), axis=1)
        hh = jnp.maximum(jnp.dot(pooled, fw1_ref[...],
                                 preferred_element_type=f32) + fb1_ref[...],
                         0.0)
        hh = jnp.maximum(jnp.dot(hh, fw2_ref[...],
                                 preferred_element_type=f32) + fb2_ref[...],
                         0.0)
        outs.append(jnp.sum(hh * fw3t_ref[...], axis=1, keepdims=True)
                    + fb3_ref[...])
    out_ref[0] = jnp.concatenate(outs, axis=0)


def kernel(site_emb_w, site_emb_b, bond_emb_w, bond_emb_b,
           conv_wsig, conv_bsig, conv_wsoft, conv_bsoft,
           fc_w1, fc_b1, fc_w2, fc_b2, fc_w3, fc_b3,
           sites_raw, bonds_raw, idx1, idx2):
    f32, bf16 = jnp.float32, jnp.bfloat16
    B, N, _ = sites_raw.shape
    E = bonds_raw.shape[1]
    S = site_emb_w.shape[1]
    C = bond_emb_w.shape[0]
    Bn = bond_emb_w.shape[1]
    L = conv_wsig.shape[0]
    sub, nchunks = _SUB, _NCHUNKS
    if B % sub != 0:
        sub, nchunks = 8, 1
    G = B // sub

    w_f = jnp.concatenate([conv_wsig, conv_wsoft], axis=-1).astype(f32)
    b_f = jnp.concatenate([conv_bsig, conv_bsoft], axis=-1).astype(f32)
    w12 = jnp.concatenate([w_f[:, :S, :], w_f[:, S:2 * S, :]], axis=2)
    w_bond = jnp.transpose(w_f[:, 2 * S:, :], (1, 0, 2)).reshape(Bn, L * 2 * S)
    a_bond = (bond_emb_w.astype(f32) @ w_bond).astype(bf16)
    c_bond = (bond_emb_b.astype(f32) @ w_bond
              + b_f.reshape(L * 2 * S))[None]

    u0 = site_emb_w.astype(f32) @ w12[0]
    c0 = site_emb_b[None].astype(f32) @ w12[0]
    u0c0 = jnp.concatenate([u0, c0], axis=0)
    se2 = jnp.concatenate([site_emb_w.astype(f32),
                           site_emb_b[None].astype(f32)], axis=0)

    mu = jnp.linspace(0.0, 10.0, C, dtype=f32)
    dmu2 = jnp.stack([jnp.ones_like(mu), -mu], axis=0)

    oh1 = jax.nn.one_hot(idx1, N, dtype=f32)
    oh2 = jax.nn.one_hot(idx2, N, dtype=f32)
    gcat = jnp.concatenate([oh1, oh2], axis=1).astype(bf16)
    scat = oh1.T.astype(bf16)

    ones_s = jnp.ones((B * N, 1), f32)
    sraw2 = jnp.concatenate([sites_raw.reshape(B * N, 1).astype(f32),
                             ones_s], axis=1).reshape(G, sub * N, 2)
    ones_d = jnp.ones((B * E, 1), f32)
    draw2 = jnp.concatenate([bonds_raw.reshape(B * E, 1).astype(f32),
                             ones_d], axis=1).reshape(G, sub * E, 2)

    h1, h2 = fc_w1.shape[1], fc_w2.shape[1]
    kern = functools.partial(_trunk_kernel, n_layers=L, n_sites=N,
                             n_edges=E, sub=sub, nchunks=nchunks)
    out = pl.pallas_call(
        kern,
        out_shape=jax.ShapeDtypeStruct((G, sub, 1), f32),
        grid=(G,),
        in_specs=[
            pl.BlockSpec((E, 2 * N), lambda g: (0, 0)),
            pl.BlockSpec((N, E), lambda g: (0, 0)),
            pl.BlockSpec((1, sub * N, 2), lambda g: (g, 0, 0)),
            pl.BlockSpec((1, sub * E, 2), lambda g: (g, 0, 0)),
            pl.BlockSpec((2, C), lambda g: (0, 0)),
            pl.BlockSpec((2, S), lambda g: (0, 0)),
            pl.BlockSpec((2, 4 * S), lambda g: (0, 0)),
            pl.BlockSpec((L, S, 4 * S), lambda g: (0, 0, 0)),
            pl.BlockSpec((C, L * 2 * S), lambda g: (0, 0)),
            pl.BlockSpec((1, L * 2 * S), lambda g: (0, 0)),
            pl.BlockSpec((S, h1), lambda g: (0, 0)),
            pl.BlockSpec((1, h1), lambda g: (0, 0)),
            pl.BlockSpec((h1, h2), lambda g: (0, 0)),
            pl.BlockSpec((1, h2), lambda g: (0, 0)),
            pl.BlockSpec((1, h2), lambda g: (0, 0)),
            pl.BlockSpec((1, 1), lambda g: (0, 0)),
        ],
        out_specs=pl.BlockSpec((1, sub, 1), lambda g: (g, 0, 0)),
        compiler_params=pltpu.CompilerParams(
            dimension_semantics=("parallel",),
            vmem_limit_bytes=100 * 1024 * 1024),
    )(gcat, scat, sraw2, draw2, dmu2, se2, u0c0,
      w12.astype(bf16), a_bond, c_bond,
      fc_w1.astype(f32), fc_b1[None].astype(f32),
      fc_w2.astype(f32), fc_b2[None].astype(f32),
      fc_w3.reshape(1, h2).astype(f32), fc_b3.reshape(1, 1).astype(f32))
    return out.reshape(B, 1)


# speedup vs baseline: 1.2796x; 1.2796x over previous
import functools

import jax
import jax.numpy as jnp
from jax.experimental import pallas as pl
from jax.experimental.pallas import tpu as pltpu

_SUB = 16

_LOG2E = 1.4426950408889634


def _trunk_kernel(gcat_ref, scat_ref, sraw_ref, draw_ref, mu_ref, sew_ref,
                  seb_ref, u0_ref, c0_ref, w12_ref, abond_ref, cb_ref,
                  fw1_ref, fb1_ref, fw2_ref, fb2_ref, fw3t_ref, fb3_ref,
                  out_ref, *, n_layers, n_sites, n_edges, sub):
    S = sew_ref.shape[1]
    S2 = 2 * S
    N, E = n_sites, n_edges
    bf16, f32 = jnp.bfloat16, jnp.float32

    gcat = gcat_ref[...]
    scat = scat_ref[...]

    x = sraw_ref[0]
    s = x * sew_ref[...] + seb_ref[...]

    d = draw_ref[0]
    gb = jnp.exp2(jnp.square(d - mu_ref[...]) * (-_LOG2E))
    zb = jnp.dot(gb.astype(bf16), abond_ref[...],
                 preferred_element_type=f32)

    for l in range(n_layers):
        if l == 0:
            proj = (x * u0_ref[...] + c0_ref[...]).astype(bf16)
        else:
            proj = jnp.dot(s.astype(bf16), w12_ref[l],
                           preferred_element_type=f32).astype(bf16)
        p = jnp.concatenate(
            [jnp.concatenate(
                [jnp.concatenate([proj[b * N:(b + 1) * N, :S2],
                                  proj[b * N:(b + 1) * N, S2:]], axis=0)
                 for b in range(sub)], axis=1),
             cb_ref[l]], axis=0)
        z_all = jnp.dot(gcat, p, preferred_element_type=f32)
        vs = []
        for b in range(sub):
            z_b = (z_all[:, b * S2:(b + 1) * S2]
                   + zb[b * E:(b + 1) * E, l * S2:(l + 1) * S2])
            sig = pl.reciprocal(1.0 + jnp.exp2(z_b[:, :S]), approx=False)
            vs.append(sig * jnp.maximum(z_b[:, S:], 0.0))
        vcat = jnp.concatenate(vs, axis=1).astype(bf16)
        delta = jnp.dot(scat, vcat, preferred_element_type=f32)
        s = s + jnp.concatenate([delta[:, b * S:(b + 1) * S]
                                 for b in range(sub)], axis=0)
    pooled = jnp.mean(s.reshape(sub, N, S), axis=1)
    hh = jnp.maximum(jnp.dot(pooled, fw1_ref[...],
                             preferred_element_type=f32) + fb1_ref[...], 0.0)
    hh = jnp.maximum(jnp.dot(hh, fw2_ref[...],
                             preferred_element_type=f32) + fb2_ref[...], 0.0)
    out_ref[0] = (jnp.sum(hh * fw3t_ref[...], axis=1, keepdims=True)
                  + fb3_ref[...])


def kernel(site_emb_w, site_emb_b, bond_emb_w, bond_emb_b,
           conv_wsig, conv_bsig, conv_wsoft, conv_bsoft,
           fc_w1, fc_b1, fc_w2, fc_b2, fc_w3, fc_b3,
           sites_raw, bonds_raw, idx1, idx2):
    f32, bf16 = jnp.float32, jnp.bfloat16
    B, N, _ = sites_raw.shape
    E = bonds_raw.shape[1]
    S = site_emb_w.shape[1]
    C = bond_emb_w.shape[0]
    Bn = bond_emb_w.shape[1]
    L = conv_wsig.shape[0]
    sub = _SUB if B % _SUB == 0 else (8 if B % 8 == 0 else 1)
    G = B // sub

    gs2 = jnp.concatenate([jnp.full((S,), -_LOG2E, f32),
                           jnp.ones((S,), f32)])

    w_f = jnp.concatenate([conv_wsig, conv_wsoft], axis=-1).astype(f32)
    b_f = jnp.concatenate([conv_bsig, conv_bsoft], axis=-1).astype(f32)
    w12 = jnp.concatenate([w_f[:, :S, :], w_f[:, S:2 * S, :]],
                          axis=2) * jnp.tile(gs2, 2)
    w_bond = (jnp.transpose(w_f[:, 2 * S:, :], (1, 0, 2))
              .reshape(Bn, L * 2 * S)) * jnp.tile(gs2, L)
    a_bond = (bond_emb_w.astype(f32) @ w_bond).astype(bf16)
    c_bond = (bond_emb_b.astype(f32) @ w_bond
              + b_f.reshape(L * 2 * S) * jnp.tile(gs2, L))
    cbt = jnp.tile(c_bond.reshape(L, 1, 2 * S), (1, 1, sub)).reshape(
        L, 1, sub * 2 * S)
    cb_hi = cbt.astype(bf16)
    cb_lo = (cbt - cb_hi.astype(f32)).astype(bf16)
    cb = jnp.concatenate(
        [cb_hi, cb_lo,
         jnp.zeros((L, 62, sub * 2 * S), bf16)], axis=1)

    u0 = site_emb_w.astype(f32) @ w12[0]
    c0 = site_emb_b[None].astype(f32) @ w12[0]

    mu = jnp.linspace(0.0, 10.0, C, dtype=f32)[None]

    oh1 = jax.nn.one_hot(idx1, N, dtype=f32)
    oh2 = jax.nn.one_hot(idx2, N, dtype=f32)
    gcat = jnp.concatenate(
        [oh1, oh2, jnp.ones((E, 2), f32), jnp.zeros((E, 62), f32)],
        axis=1).astype(bf16)
    scat = oh1.T.astype(bf16)

    sraw = sites_raw.reshape(G, sub * N, 1).astype(f32)
    draw = bonds_raw.reshape(G, sub * E, 1).astype(f32)

    h1, h2 = fc_w1.shape[1], fc_w2.shape[1]
    kern = functools.partial(_trunk_kernel, n_layers=L, n_sites=N,
                             n_edges=E, sub=sub)
    out = pl.pallas_call(
        kern,
        out_shape=jax.ShapeDtypeStruct((G, sub, 1), f32),
        grid=(G,),
        in_specs=[
            pl.BlockSpec((E, 2 * N + 64), lambda g: (0, 0)),
            pl.BlockSpec((N, E), lambda g: (0, 0)),
            pl.BlockSpec((1, sub * N, 1), lambda g: (g, 0, 0)),
            pl.BlockSpec((1, sub * E, 1), lambda g: (g, 0, 0)),
            pl.BlockSpec((1, C), lambda g: (0, 0)),
            pl.BlockSpec((1, S), lambda g: (0, 0)),
            pl.BlockSpec((1, S), lambda g: (0, 0)),
            pl.BlockSpec((1, 4 * S), lambda g: (0, 0)),
            pl.BlockSpec((1, 4 * S), lambda g: (0, 0)),
            pl.BlockSpec((L, S, 4 * S), lambda g: (0, 0, 0)),
            pl.BlockSpec((C, L * 2 * S), lambda g: (0, 0)),
            pl.BlockSpec((L, 64, sub * 2 * S), lambda g: (0, 0, 0)),
            pl.BlockSpec((S, h1), lambda g: (0, 0)),
            pl.BlockSpec((1, h1), lambda g: (0, 0)),
            pl.BlockSpec((h1, h2), lambda g: (0, 0)),
            pl.BlockSpec((1, h2), lambda g: (0, 0)),
            pl.BlockSpec((1, h2), lambda g: (0, 0)),
            pl.BlockSpec((1, 1), lambda g: (0, 0)),
        ],
        out_specs=pl.BlockSpec((1, sub, 1), lambda g: (g, 0, 0)),
        compiler_params=pltpu.CompilerParams(
            dimension_semantics=("parallel",),
            vmem_limit_bytes=100 * 1024 * 1024),
    )(gcat, scat, sraw, draw, mu,
      site_emb_w.astype(f32), site_emb_b[None].astype(f32),
      u0, c0, w12.astype(bf16), a_bond, cb,
      fc_w1.astype(f32), fc_b1[None].astype(f32),
      fc_w2.astype(f32), fc_b2[None].astype(f32),
      fc_w3.reshape(1, h2).astype(f32), fc_b3.reshape(1, 1).astype(f32))
    return out.reshape(B, 1)


# speedup vs baseline: 1.3344x; 1.0428x over previous
import functools

import jax
import jax.numpy as jnp
from jax.experimental import pallas as pl
from jax.experimental.pallas import tpu as pltpu

_SUB = 16

_LOG2E = 1.4426950408889634


def _head_kernel(x_ref, fw1_ref, fb1_ref, fw2_ref, fb2_ref, fw3t_ref,
                 fb3_ref, out_ref):
    f32 = jnp.float32
    hh = jnp.maximum(jnp.dot(x_ref[...], fw1_ref[...],
                             preferred_element_type=f32) + fb1_ref[...], 0.0)
    hh = jnp.maximum(jnp.dot(hh, fw2_ref[...],
                             preferred_element_type=f32) + fb2_ref[...], 0.0)
    out_ref[...] = (jnp.sum(hh * fw3t_ref[...], axis=1, keepdims=True)
                    + fb3_ref[...])


def _trunk_kernel(gcat_ref, scat_ref, sraw_ref, draw_ref, mu_ref, sew_ref,
                  seb_ref, u0_ref, c0_ref, w12_ref, abond_ref, cb_ref,
                  out_ref, *, n_layers, n_sites, n_edges, sub):
    S = sew_ref.shape[1]
    S2 = 2 * S
    N, E = n_sites, n_edges
    bf16, f32 = jnp.bfloat16, jnp.float32

    gcat = gcat_ref[...]
    scat = scat_ref[...]

    x = sraw_ref[0]
    s = x * sew_ref[...] + seb_ref[...]

    d = draw_ref[0]
    gb = jnp.exp2(jnp.square(d - mu_ref[...]) * (-_LOG2E))
    zb = jnp.dot(gb.astype(bf16), abond_ref[...],
                 preferred_element_type=f32)

    for l in range(n_layers):
        if l == 0:
            proj = (x * u0_ref[...] + c0_ref[...]).astype(bf16)
        else:
            proj = jnp.dot(s.astype(bf16), w12_ref[l],
                           preferred_element_type=f32).astype(bf16)
        p = jnp.concatenate(
            [jnp.concatenate(
                [jnp.concatenate([proj[b * N:(b + 1) * N, :S2],
                                  proj[b * N:(b + 1) * N, S2:]], axis=0)
                 for b in range(sub)], axis=1),
             cb_ref[l]], axis=0)
        z_all = jnp.dot(gcat, p, preferred_element_type=f32)
        vs = []
        for b in range(sub):
            z_b = (z_all[:, b * S2:(b + 1) * S2]
                   + zb[b * E:(b + 1) * E, l * S2:(l + 1) * S2])
            sig = pl.reciprocal(1.0 + jnp.exp2(z_b[:, :S]), approx=False)
            vs.append(sig * jnp.maximum(z_b[:, S:], 0.0))
        vcat = jnp.concatenate(vs, axis=1).astype(bf16)
        delta = jnp.dot(scat, vcat, preferred_element_type=f32)
        s = s + jnp.concatenate([delta[:, b * S:(b + 1) * S]
                                 for b in range(sub)], axis=0)
    out_ref[0] = jnp.mean(s.reshape(sub, N, S), axis=1)


def kernel(site_emb_w, site_emb_b, bond_emb_w, bond_emb_b,
           conv_wsig, conv_bsig, conv_wsoft, conv_bsoft,
           fc_w1, fc_b1, fc_w2, fc_b2, fc_w3, fc_b3,
           sites_raw, bonds_raw, idx1, idx2):
    f32, bf16 = jnp.float32, jnp.bfloat16
    B, N, _ = sites_raw.shape
    E = bonds_raw.shape[1]
    S = site_emb_w.shape[1]
    C = bond_emb_w.shape[0]
    Bn = bond_emb_w.shape[1]
    L = conv_wsig.shape[0]
    sub = _SUB if B % _SUB == 0 else (8 if B % 8 == 0 else 1)
    G = B // sub

    gs2 = jnp.concatenate([jnp.full((S,), -_LOG2E, f32),
                           jnp.ones((S,), f32)])

    w_f = jnp.concatenate([conv_wsig, conv_wsoft], axis=-1).astype(f32)
    b_f = jnp.concatenate([conv_bsig, conv_bsoft], axis=-1).astype(f32)
    w12 = jnp.concatenate([w_f[:, :S, :], w_f[:, S:2 * S, :]],
                          axis=2) * jnp.tile(gs2, 2)
    w_bond = (jnp.transpose(w_f[:, 2 * S:, :], (1, 0, 2))
              .reshape(Bn, L * 2 * S)) * jnp.tile(gs2, L)
    a_bond = (bond_emb_w.astype(f32) @ w_bond).astype(bf16)
    c_bond = (bond_emb_b.astype(f32) @ w_bond
              + b_f.reshape(L * 2 * S) * jnp.tile(gs2, L))
    cbt = jnp.tile(c_bond.reshape(L, 1, 2 * S), (1, 1, sub)).reshape(
        L, 1, sub * 2 * S)
    cb_hi = cbt.astype(bf16)
    cb_lo = (cbt - cb_hi.astype(f32)).astype(bf16)
    cb = jnp.concatenate(
        [cb_hi, cb_lo,
         jnp.zeros((L, 62, sub * 2 * S), bf16)], axis=1)

    u0 = site_emb_w.astype(f32) @ w12[0]
    c0 = site_emb_b[None].astype(f32) @ w12[0]

    mu = jnp.linspace(0.0, 10.0, C, dtype=f32)[None]

    oh1 = jax.nn.one_hot(idx1, N, dtype=f32)
    oh2 = jax.nn.one_hot(idx2, N, dtype=f32)
    gcat = jnp.concatenate(
        [oh1, oh2, jnp.ones((E, 2), f32), jnp.zeros((E, 62), f32)],
        axis=1).astype(bf16)
    scat = oh1.T.astype(bf16)

    sraw = sites_raw.reshape(G, sub * N, 1).astype(f32)
    draw = bonds_raw.reshape(G, sub * E, 1).astype(f32)

    h1, h2 = fc_w1.shape[1], fc_w2.shape[1]
    kern = functools.partial(_trunk_kernel, n_layers=L, n_sites=N,
                             n_edges=E, sub=sub)
    pooled = pl.pallas_call(
        kern,
        out_shape=jax.ShapeDtypeStruct((G, sub, S), f32),
        grid=(G,),
        in_specs=[
            pl.BlockSpec((E, 2 * N + 64), lambda g: (0, 0)),
            pl.BlockSpec((N, E), lambda g: (0, 0)),
            pl.BlockSpec((1, sub * N, 1), lambda g: (g, 0, 0)),
            pl.BlockSpec((1, sub * E, 1), lambda g: (g, 0, 0)),
            pl.BlockSpec((1, C), lambda g: (0, 0)),
            pl.BlockSpec((1, S), lambda g: (0, 0)),
            pl.BlockSpec((1, S), lambda g: (0, 0)),
            pl.BlockSpec((1, 4 * S), lambda g: (0, 0)),
            pl.BlockSpec((1, 4 * S), lambda g: (0, 0)),
            pl.BlockSpec((L, S, 4 * S), lambda g: (0, 0, 0)),
            pl.BlockSpec((C, L * 2 * S), lambda g: (0, 0)),
            pl.BlockSpec((L, 64, sub * 2 * S), lambda g: (0, 0, 0)),
        ],
        out_specs=pl.BlockSpec((1, sub, S), lambda g: (g, 0, 0)),
        compiler_params=pltpu.CompilerParams(
            dimension_semantics=("parallel",),
            vmem_limit_bytes=100 * 1024 * 1024),
    )(gcat, scat, sraw, draw, mu,
      site_emb_w.astype(f32), site_emb_b[None].astype(f32),
      u0, c0, w12.astype(bf16), a_bond, cb)

    rows = 4096 if B % 4096 == 0 else B
    out = pl.pallas_call(
        _head_kernel,
        out_shape=jax.ShapeDtypeStruct((B, 1), f32),
        grid=(B // rows,),
        in_specs=[
            pl.BlockSpec((rows, S), lambda g: (g, 0)),
            pl.BlockSpec((S, h1), lambda g: (0, 0)),
            pl.BlockSpec((1, h1), lambda g: (0, 0)),
            pl.BlockSpec((h1, h2), lambda g: (0, 0)),
            pl.BlockSpec((1, h2), lambda g: (0, 0)),
            pl.BlockSpec((1, h2), lambda g: (0, 0)),
            pl.BlockSpec((1, 1), lambda g: (0, 0)),
        ],
        out_specs=pl.BlockSpec((rows, 1), lambda g: (g, 0)),
        compiler_params=pltpu.CompilerParams(
            dimension_semantics=("parallel",)),
    )(pooled.reshape(B, S),
      fc_w1.astype(f32), fc_b1[None].astype(f32),
      fc_w2.astype(f32), fc_b2[None].astype(f32),
      fc_w3.reshape(1, h2).astype(f32), fc_b3.reshape(1, 1).astype(f32))
    return out


# speedup vs baseline: 1.7401x; 1.3041x over previous
import functools

import jax
import jax.numpy as jnp
from jax.experimental import pallas as pl
from jax.experimental.pallas import tpu as pltpu

_SUB = 16

_LOG2E = 1.4426950408889634


def _head_kernel(x_ref, fw1_ref, fb1_ref, fw2_ref, fb2_ref, fw3t_ref,
                 fb3_ref, out_ref):
    f32 = jnp.float32
    hh = jnp.maximum(jnp.dot(x_ref[...], fw1_ref[...],
                             preferred_element_type=f32) + fb1_ref[...], 0.0)
    hh = jnp.maximum(jnp.dot(hh, fw2_ref[...],
                             preferred_element_type=f32) + fb2_ref[...], 0.0)
    out_ref[...] = (jnp.sum(hh * fw3t_ref[...], axis=1, keepdims=True)
                    + fb3_ref[...])


def _trunk_kernel(gcat_ref, scat_ref, sraw_ref, draw_ref, mu_ref, sew_ref,
                  seb_ref, u0_ref, c0_ref, w12_ref, abond_ref, cb_ref,
                  out_ref, *, n_layers, n_sites, n_edges, sub):
    S = sew_ref.shape[1]
    S2 = 2 * S
    N, E = n_sites, n_edges
    bf16, f32 = jnp.bfloat16, jnp.float32

    gcat = gcat_ref[...]
    scat = scat_ref[...]

    xT = jnp.transpose(sraw_ref[0])
    dT = jnp.transpose(draw_ref[0])
    s = jnp.concatenate(
        [xT[:, b:b + 1] * sew_ref[...] + seb_ref[...]
         for b in range(sub)], axis=0)
    gb = jnp.concatenate(
        [jnp.exp2(jnp.square(dT[:, b:b + 1] - mu_ref[...]) * (-_LOG2E))
         for b in range(sub)], axis=0)
    zb = jnp.dot(gb.astype(bf16), abond_ref[...],
                 preferred_element_type=f32)

    for l in range(n_layers):
        if l == 0:
            proj = jnp.concatenate(
                [xT[:, b:b + 1] * u0_ref[...] + c0_ref[...]
                 for b in range(sub)], axis=0).astype(bf16)
        else:
            proj = jnp.dot(s.astype(bf16), w12_ref[l],
                           preferred_element_type=f32).astype(bf16)
        p = jnp.concatenate(
            [jnp.concatenate(
                [jnp.concatenate([proj[b * N:(b + 1) * N, :S2],
                                  proj[b * N:(b + 1) * N, S2:]], axis=0)
                 for b in range(sub)], axis=1),
             cb_ref[l]], axis=0)
        z_all = jnp.dot(gcat, p, preferred_element_type=f32)
        vs = []
        for b in range(sub):
            z_b = (z_all[:, b * S2:(b + 1) * S2]
                   + zb[b * E:(b + 1) * E, l * S2:(l + 1) * S2])
            sig = pl.reciprocal(1.0 + jnp.exp2(z_b[:, :S]), approx=False)
            vs.append(sig * jnp.maximum(z_b[:, S:], 0.0))
        vcat = jnp.concatenate(vs, axis=1).astype(bf16)
        delta = jnp.dot(scat, vcat, preferred_element_type=f32)
        s = s + jnp.concatenate([delta[:, b * S:(b + 1) * S]
                                 for b in range(sub)], axis=0)
    out_ref[0] = jnp.mean(s.reshape(sub, N, S), axis=1)


def kernel(site_emb_w, site_emb_b, bond_emb_w, bond_emb_b,
           conv_wsig, conv_bsig, conv_wsoft, conv_bsoft,
           fc_w1, fc_b1, fc_w2, fc_b2, fc_w3, fc_b3,
           sites_raw, bonds_raw, idx1, idx2):
    f32, bf16 = jnp.float32, jnp.bfloat16
    B, N, _ = sites_raw.shape
    E = bonds_raw.shape[1]
    S = site_emb_w.shape[1]
    C = bond_emb_w.shape[0]
    Bn = bond_emb_w.shape[1]
    L = conv_wsig.shape[0]
    sub = _SUB if B % _SUB == 0 else (8 if B % 8 == 0 else 1)
    G = B // sub

    gs2 = jnp.concatenate([jnp.full((S,), -_LOG2E, f32),
                           jnp.ones((S,), f32)])

    w_f = jnp.concatenate([conv_wsig, conv_wsoft], axis=-1).astype(f32)
    b_f = jnp.concatenate([conv_bsig, conv_bsoft], axis=-1).astype(f32)
    w12 = jnp.concatenate([w_f[:, :S, :], w_f[:, S:2 * S, :]],
                          axis=2) * jnp.tile(gs2, 2)
    w_bond = (jnp.transpose(w_f[:, 2 * S:, :], (1, 0, 2))
              .reshape(Bn, L * 2 * S)) * jnp.tile(gs2, L)
    a_bond = (bond_emb_w.astype(f32) @ w_bond).astype(bf16)
    c_bond = (bond_emb_b.astype(f32) @ w_bond
              + b_f.reshape(L * 2 * S) * jnp.tile(gs2, L))
    cbt = jnp.tile(c_bond.reshape(L, 1, 2 * S), (1, 1, sub)).reshape(
        L, 1, sub * 2 * S)
    cb_hi = cbt.astype(bf16)
    cb_lo = (cbt - cb_hi.astype(f32)).astype(bf16)
    cb = jnp.concatenate(
        [cb_hi, cb_lo,
         jnp.zeros((L, 62, sub * 2 * S), bf16)], axis=1)

    u0 = site_emb_w.astype(f32) @ w12[0]
    c0 = site_emb_b[None].astype(f32) @ w12[0]

    mu = jnp.linspace(0.0, 10.0, C, dtype=f32)[None]

    oh1 = jax.nn.one_hot(idx1, N, dtype=f32)
    oh2 = jax.nn.one_hot(idx2, N, dtype=f32)
    gcat = jnp.concatenate(
        [oh1, oh2, jnp.ones((E, 2), f32), jnp.zeros((E, 62), f32)],
        axis=1).astype(bf16)
    scat = oh1.T.astype(bf16)

    sraw = sites_raw.reshape(G, sub, N).astype(f32)
    draw = bonds_raw.reshape(G, sub, E).astype(f32)

    h1, h2 = fc_w1.shape[1], fc_w2.shape[1]
    kern = functools.partial(_trunk_kernel, n_layers=L, n_sites=N,
                             n_edges=E, sub=sub)
    pooled = pl.pallas_call(
        kern,
        out_shape=jax.ShapeDtypeStruct((G, sub, S), f32),
        grid=(G,),
        in_specs=[
            pl.BlockSpec((E, 2 * N + 64), lambda g: (0, 0)),
            pl.BlockSpec((N, E), lambda g: (0, 0)),
            pl.BlockSpec((1, sub, N), lambda g: (g, 0, 0)),
            pl.BlockSpec((1, sub, E), lambda g: (g, 0, 0)),
            pl.BlockSpec((1, C), lambda g: (0, 0)),
            pl.BlockSpec((1, S), lambda g: (0, 0)),
            pl.BlockSpec((1, S), lambda g: (0, 0)),
            pl.BlockSpec((1, 4 * S), lambda g: (0, 0)),
            pl.BlockSpec((1, 4 * S), lambda g: (0, 0)),
            pl.BlockSpec((L, S, 4 * S), lambda g: (0, 0, 0)),
            pl.BlockSpec((C, L * 2 * S), lambda g: (0, 0)),
            pl.BlockSpec((L, 64, sub * 2 * S), lambda g: (0, 0, 0)),
        ],
        out_specs=pl.BlockSpec((1, sub, S), lambda g: (g, 0, 0)),
        compiler_params=pltpu.CompilerParams(
            dimension_semantics=("parallel",),
            vmem_limit_bytes=100 * 1024 * 1024),
    )(gcat, scat, sraw, draw, mu,
      site_emb_w.astype(f32), site_emb_b[None].astype(f32),
      u0, c0, w12.astype(bf16), a_bond, cb)

    rows = 4096 if B % 4096 == 0 else B
    out = pl.pallas_call(
        _head_kernel,
        out_shape=jax.ShapeDtypeStruct((B, 1), f32),
        grid=(B // rows,),
        in_specs=[
            pl.BlockSpec((rows, S), lambda g: (g, 0)),
            pl.BlockSpec((S, h1), lambda g: (0, 0)),
            pl.BlockSpec((1, h1), lambda g: (0, 0)),
            pl.BlockSpec((h1, h2), lambda g: (0, 0)),
            pl.BlockSpec((1, h2), lambda g: (0, 0)),
            pl.BlockSpec((1, h2), lambda g: (0, 0)),
            pl.BlockSpec((1, 1), lambda g: (0, 0)),
        ],
        out_specs=pl.BlockSpec((rows, 1), lambda g: (g, 0)),
        compiler_params=pltpu.CompilerParams(
            dimension_semantics=("parallel",)),
    )(pooled.reshape(B, S),
      fc_w1.astype(f32), fc_b1[None].astype(f32),
      fc_w2.astype(f32), fc_b2[None].astype(f32),
      fc_w3.reshape(1, h2).astype(f32), fc_b3.reshape(1, 1).astype(f32))
    return out


# speedup vs baseline: 1.7588x; 1.0107x over previous
import functools

import jax
import jax.numpy as jnp
from jax.experimental import pallas as pl
from jax.experimental.pallas import tpu as pltpu

_SUB = 16

_LOG2E = 1.4426950408889634


def _head_kernel(x_ref, fw1_ref, fb1_ref, fw2_ref, fb2_ref, fw3t_ref,
                 fb3_ref, out_ref):
    f32 = jnp.float32
    hh = jnp.maximum(jnp.dot(x_ref[...], fw1_ref[...],
                             preferred_element_type=f32) + fb1_ref[...], 0.0)
    hh = jnp.maximum(jnp.dot(hh, fw2_ref[...],
                             preferred_element_type=f32) + fb2_ref[...], 0.0)
    out_ref[...] = (jnp.sum(hh * fw3t_ref[...], axis=1, keepdims=True)
                    + fb3_ref[...])


def _trunk_kernel(gcat_ref, scat_ref, sraw_ref, draw_ref, mu_ref, sew_ref,
                  seb_ref, u0_ref, c0_ref, w12_ref, abond_ref, cb_ref,
                  out_ref, *, n_layers, n_sites, n_edges, sub):
    S = sew_ref.shape[1]
    S2 = 2 * S
    N, E = n_sites, n_edges
    bf16, f32 = jnp.bfloat16, jnp.float32

    gcat = gcat_ref[...]
    scat = scat_ref[...]

    xT = jnp.transpose(sraw_ref[0])
    dT = jnp.transpose(draw_ref[0])
    s = jnp.concatenate(
        [xT[:, b:b + 1] * sew_ref[...] + seb_ref[...]
         for b in range(sub)], axis=0)
    gb = jnp.concatenate(
        [jnp.exp2(jnp.square(dT[:, b:b + 1] - mu_ref[...]) * (-_LOG2E))
         for b in range(sub)], axis=0)
    zb = jnp.dot(gb, abond_ref[...],
                 preferred_element_type=f32)

    for l in range(n_layers):
        if l == 0:
            proj = jnp.concatenate(
                [xT[:, b:b + 1] * u0_ref[...] + c0_ref[...]
                 for b in range(sub)], axis=0).astype(bf16)
        else:
            proj = jnp.dot(s.astype(bf16), w12_ref[l],
                           preferred_element_type=f32).astype(bf16)
        p = jnp.concatenate(
            [jnp.concatenate(
                [jnp.concatenate([proj[b * N:(b + 1) * N, :S2],
                                  proj[b * N:(b + 1) * N, S2:]], axis=0)
                 for b in range(sub)], axis=1),
             cb_ref[l]], axis=0)
        z_all = jnp.dot(gcat, p, preferred_element_type=f32)
        vs = []
        for b in range(sub):
            z_b = (z_all[:, b * S2:(b + 1) * S2]
                   + zb[b * E:(b + 1) * E, l * S2:(l + 1) * S2])
            sig = pl.reciprocal(1.0 + jnp.exp2(z_b[:, :S]), approx=False)
            vs.append(sig * jnp.maximum(z_b[:, S:], 0.0))
        vcat = jnp.concatenate(vs, axis=1)
        delta = jnp.dot(scat, vcat, preferred_element_type=f32)
        s = s + jnp.concatenate([delta[:, b * S:(b + 1) * S]
                                 for b in range(sub)], axis=0)
    out_ref[0] = jnp.mean(s.reshape(sub, N, S), axis=1)


def kernel(site_emb_w, site_emb_b, bond_emb_w, bond_emb_b,
           conv_wsig, conv_bsig, conv_wsoft, conv_bsoft,
           fc_w1, fc_b1, fc_w2, fc_b2, fc_w3, fc_b3,
           sites_raw, bonds_raw, idx1, idx2):
    f32, bf16 = jnp.float32, jnp.bfloat16
    B, N, _ = sites_raw.shape
    E = bonds_raw.shape[1]
    S = site_emb_w.shape[1]
    C = bond_emb_w.shape[0]
    Bn = bond_emb_w.shape[1]
    L = conv_wsig.shape[0]
    sub = _SUB if B % _SUB == 0 else (8 if B % 8 == 0 else 1)
    G = B // sub

    gs2 = jnp.concatenate([jnp.full((S,), -_LOG2E, f32),
                           jnp.ones((S,), f32)])

    w_f = jnp.concatenate([conv_wsig, conv_wsoft], axis=-1).astype(f32)
    b_f = jnp.concatenate([conv_bsig, conv_bsoft], axis=-1).astype(f32)
    w12 = jnp.concatenate([w_f[:, :S, :], w_f[:, S:2 * S, :]],
                          axis=2) * jnp.tile(gs2, 2)
    w_bond = (jnp.transpose(w_f[:, 2 * S:, :], (1, 0, 2))
              .reshape(Bn, L * 2 * S)) * jnp.tile(gs2, L)
    a_bond = bond_emb_w.astype(f32) @ w_bond
    c_bond = (bond_emb_b.astype(f32) @ w_bond
              + b_f.reshape(L * 2 * S) * jnp.tile(gs2, L))
    cbt = jnp.tile(c_bond.reshape(L, 1, 2 * S), (1, 1, sub)).reshape(
        L, 1, sub * 2 * S)
    cb_hi = cbt.astype(bf16)
    cb_lo = (cbt - cb_hi.astype(f32)).astype(bf16)
    cb = jnp.concatenate(
        [cb_hi, cb_lo,
         jnp.zeros((L, 62, sub * 2 * S), bf16)], axis=1)

    u0 = site_emb_w.astype(f32) @ w12[0]
    c0 = site_emb_b[None].astype(f32) @ w12[0]

    mu = jnp.linspace(0.0, 10.0, C, dtype=f32)[None]

    oh1 = jax.nn.one_hot(idx1, N, dtype=f32)
    oh2 = jax.nn.one_hot(idx2, N, dtype=f32)
    gcat = jnp.concatenate(
        [oh1, oh2, jnp.ones((E, 2), f32), jnp.zeros((E, 62), f32)],
        axis=1).astype(bf16)
    scat = oh1.T

    sraw = sites_raw.reshape(G, sub, N).astype(f32)
    draw = bonds_raw.reshape(G, sub, E).astype(f32)

    h1, h2 = fc_w1.shape[1], fc_w2.shape[1]
    kern = functools.partial(_trunk_kernel, n_layers=L, n_sites=N,
                             n_edges=E, sub=sub)
    pooled = pl.pallas_call(
        kern,
        out_shape=jax.ShapeDtypeStruct((G, sub, S), f32),
        grid=(G,),
        in_specs=[
            pl.BlockSpec((E, 2 * N + 64), lambda g: (0, 0)),
            pl.BlockSpec((N, E), lambda g: (0, 0)),
            pl.BlockSpec((1, sub, N), lambda g: (g, 0, 0)),
            pl.BlockSpec((1, sub, E), lambda g: (g, 0, 0)),
            pl.BlockSpec((1, C), lambda g: (0, 0)),
            pl.BlockSpec((1, S), lambda g: (0, 0)),
            pl.BlockSpec((1, S), lambda g: (0, 0)),
            pl.BlockSpec((1, 4 * S), lambda g: (0, 0)),
            pl.BlockSpec((1, 4 * S), lambda g: (0, 0)),
            pl.BlockSpec((L, S, 4 * S), lambda g: (0, 0, 0)),
            pl.BlockSpec((C, L * 2 * S), lambda g: (0, 0)),
            pl.BlockSpec((L, 64, sub * 2 * S), lambda g: (0, 0, 0)),
        ],
        out_specs=pl.BlockSpec((1, sub, S), lambda g: (g, 0, 0)),
        compiler_params=pltpu.CompilerParams(
            dimension_semantics=("parallel",),
            vmem_limit_bytes=100 * 1024 * 1024),
    )(gcat, scat, sraw, draw, mu,
      site_emb_w.astype(f32), site_emb_b[None].astype(f32),
      u0, c0, w12.astype(bf16), a_bond, cb)

    rows = 4096 if B % 4096 == 0 else B
    out = pl.pallas_call(
        _head_kernel,
        out_shape=jax.ShapeDtypeStruct((B, 1), f32),
        grid=(B // rows,),
        in_specs=[
            pl.BlockSpec((rows, S), lambda g: (g, 0)),
            pl.BlockSpec((S, h1), lambda g: (0, 0)),
            pl.BlockSpec((1, h1), lambda g: (0, 0)),
            pl.BlockSpec((h1, h2), lambda g: (0, 0)),
            pl.BlockSpec((1, h2), lambda g: (0, 0)),
            pl.BlockSpec((1, h2), lambda g: (0, 0)),
            pl.BlockSpec((1, 1), lambda g: (0, 0)),
        ],
        out_specs=pl.BlockSpec((rows, 1), lambda g: (g, 0)),
        compiler_params=pltpu.CompilerParams(
            dimension_semantics=("parallel",)),
    )(pooled.reshape(B, S),
      fc_w1.astype(f32), fc_b1[None].astype(f32),
      fc_w2.astype(f32), fc_b2[None].astype(f32),
      fc_w3.reshape(1, h2).astype(f32), fc_b3.reshape(1, 1).astype(f32))
    return out
